# Initial kernel scaffold; baseline (speedup 1.0000x reference)
import jax
import jax.numpy as jnp
from jax.experimental import pallas as pl


def kernel(x, edge_index, batch, W1, b1, W2, b2, W3, b3):
    raise NotImplementedError("write your pallas kernel here")


# speedup vs baseline: 1.9986x; 1.9986x over previous
import jax
import jax.numpy as jnp
from jax import lax
from jax.experimental import pallas as pl
from jax.experimental.pallas import tpu as pltpu
from jax.experimental.pallas import tpu_sc as plsc

N = 10000
E = 160000
G = 128
D_IN = 256
D_H = 512
D_OUT = 256

NPAD = 10240
EPAD = 163840
RPT = NPAD // 16
NB = EPAD // 128 // 16
GRP = 2
CW = 128
CN = D_H // CW
NHALF = NPAD // 2
HPT = NHALF // 16
ACCROWS = NHALF + 8
R = 1280

_MESH = plsc.VectorSubcoreMesh(core_axis_name="c", subcore_axis_name="s")


def _sc_count(dst2d, zeros_h, ones_h):
  def body(dst_hbm, z_hbm, ones_hbm, out_hbm, dstv, ones_v, acc):
    c = lax.axis_index("c")
    s = lax.axis_index("s")
    base = c * NHALF
    pltpu.sync_copy(dst_hbm.at[pl.ds(s * NB, NB)], dstv)
    pltpu.sync_copy(ones_hbm, ones_v)
    pltpu.sync_copy(z_hbm.at[pl.ds(s * HPT, HPT)], acc.at[pl.ds(s * HPT, HPT)])

    def remap(j, carry):
      for l in range(8):
        dv = dstv[j, pl.ds(l * 16, 16)] - base
        oob = (dv < 0) | (dv >= NHALF)
        dstv[j, pl.ds(l * 16, 16)] = jnp.where(oob, NHALF, dv)
      return carry

    lax.fori_loop(0, NB, remap, 0)
    plsc.subcore_barrier()

    def step(j, carry):
      pltpu.sync_copy(ones_v, acc.at[dstv.at[j]], add=True)
      return carry

    lax.fori_loop(0, NB, step, 0)
    plsc.subcore_barrier()
    pltpu.sync_copy(acc.at[pl.ds(s * HPT, HPT)],
                    out_hbm.at[pl.ds(base + s * HPT, HPT)])

  kfn = pl.kernel(
      body,
      out_type=jax.ShapeDtypeStruct((NPAD, 128), jnp.float32),
      mesh=_MESH,
      scratch_types=[
          pltpu.VMEM((NB, 128), jnp.int32),
          pltpu.VMEM((128, 128), jnp.float32),
          pltpu.VMEM_SHARED((ACCROWS, 128), jnp.float32),
      ],
  )
  return kfn(dst2d, zeros_h, ones_h)


def _sc_aggregate(u, src2d, dst2d):
  def body(u_hbm, src_hbm, dst_hbm, out_hbm, srcv, dstv, rows, acc, gsem,
           ssem):
    c = lax.axis_index("c")
    s = lax.axis_index("s")
    base = c * NHALF
    pltpu.sync_copy(src_hbm.at[pl.ds(s * NB, NB)], srcv)
    pltpu.sync_copy(dst_hbm.at[pl.ds(s * NB, NB)], dstv)

    def remap(j, carry):
      for l in range(8):
        d = dstv[j, pl.ds(l * 16, 16)] - base
        oob = (d < 0) | (d >= NHALF)
        dstv[j, pl.ds(l * 16, 16)] = jnp.where(oob, NHALF, d)
      return carry

    lax.fori_loop(0, NB, remap, 0)

    for kk in range(CN):
      pltpu.sync_copy(u_hbm.at[kk, pl.ds(base + s * HPT, HPT)],
                      acc.at[pl.ds(s * HPT, HPT)])
      plsc.subcore_barrier()

      def group(g, carry):
        j0 = g * GRP
        cps = [
            pltpu.async_copy(u_hbm.at[kk].at[srcv.at[j0 + b]],
                             rows.at[b], gsem)
            for b in range(GRP)
        ]
        for cp in cps:
          cp.wait()
        scs = [
            pltpu.async_copy(rows.at[b], acc.at[dstv.at[j0 + b]], ssem,
                             add=True)
            for b in range(GRP)
        ]
        for sc in scs:
          sc.wait()
        return carry

      lax.fori_loop(0, NB // GRP, group, 0)
      plsc.subcore_barrier()
      pltpu.sync_copy(acc.at[pl.ds(s * HPT, HPT)],
                      out_hbm.at[kk, pl.ds(base + s * HPT, HPT)])
      plsc.subcore_barrier()

  kfn = pl.kernel(
      body,
      out_type=jax.ShapeDtypeStruct((CN, NPAD, CW), jnp.float32),
      mesh=_MESH,
      scratch_types=[
          pltpu.VMEM((NB, 128), jnp.int32),
          pltpu.VMEM((NB, 128), jnp.int32),
          pltpu.VMEM((GRP, 128, CW), jnp.float32),
          pltpu.VMEM_SHARED((ACCROWS, CW), jnp.float32),
          pltpu.SemaphoreType.DMA,
          pltpu.SemaphoreType.DMA,
      ],
  )
  return kfn(u, src2d, dst2d)


def _tc_first(x_p, W1, cnt2):
  def body(x_ref, w_ref, cnt_ref, u_ref, dinv_ref):
    cnt = cnt_ref[:, :1]
    dinv = lax.rsqrt(cnt + 1.0)
    xw = jnp.dot(x_ref[:], w_ref[:], preferred_element_type=jnp.float32)
    u_ref[0] = xw * dinv
    dinv_ref[:] = dinv

  return pl.pallas_call(
      body,
      grid=(NPAD // R, CN),
      in_specs=[
          pl.BlockSpec((R, D_IN), lambda r, co: (r, 0)),
          pl.BlockSpec((D_IN, CW), lambda r, co: (0, co)),
          pl.BlockSpec((R, 128), lambda r, co: (r, 0)),
      ],
      out_specs=[
          pl.BlockSpec((1, R, CW), lambda r, co: (co, r, 0)),
          pl.BlockSpec((R, 1), lambda r, co: (r, 0)),
      ],
      out_shape=[
          jax.ShapeDtypeStruct((CN, NPAD, CW), jnp.float32),
          jax.ShapeDtypeStruct((NPAD, 1), jnp.float32),
      ],
  )(x_p, W1, cnt2)


def _tc_mid(agg, dinv, b_row, W):
  def body(agg_ref, dinv_ref, b_ref, w_ref, u_ref):
    dinv = dinv_ref[:]
    acc = jnp.zeros((R, CW), dtype=jnp.float32)
    for k2 in range(CN):
      hk = jnp.maximum(agg_ref[k2] * dinv + b_ref[:, k2 * CW:(k2 + 1) * CW],
                       0.0)
      acc += jnp.dot(hk, w_ref[k2 * CW:(k2 + 1) * CW, :],
                     preferred_element_type=jnp.float32)
    u_ref[0] = acc * dinv

  return pl.pallas_call(
      body,
      grid=(NPAD // R, CN),
      in_specs=[
          pl.BlockSpec((CN, R, CW), lambda r, co: (0, r, 0)),
          pl.BlockSpec((R, 1), lambda r, co: (r, 0)),
          pl.BlockSpec((1, D_H), lambda r, co: (0, 0)),
          pl.BlockSpec((D_H, CW), lambda r, co: (0, co)),
      ],
      out_specs=pl.BlockSpec((1, R, CW), lambda r, co: (co, r, 0)),
      out_shape=jax.ShapeDtypeStruct((CN, NPAD, CW), jnp.float32),
  )(agg, dinv, b_row, W)


def _tc_relu(agg, dinv, b_row):
  CO = D_OUT // CW

  def body(agg_ref, dinv_ref, b_ref, h_ref):
    full = jnp.concatenate([agg_ref[k] for k in range(CO)], axis=1)
    h_ref[:] = jnp.maximum(full * dinv_ref[:] + b_ref[:], 0.0)

  return pl.pallas_call(
      body,
      grid=(NPAD // R,),
      in_specs=[
          pl.BlockSpec((CO, R, CW), lambda r: (0, r, 0)),
          pl.BlockSpec((R, 1), lambda r: (r, 0)),
          pl.BlockSpec((1, D_OUT), lambda r: (0, 0)),
      ],
      out_specs=pl.BlockSpec((R, D_OUT), lambda r: (r, 0)),
      out_shape=jax.ShapeDtypeStruct((NPAD, D_OUT), jnp.float32),
  )(agg, dinv, b_row)


def _tc_finalize(h3, batch_row):
  def body(h_ref, batch_ref, out_ref):
    ids = lax.broadcasted_iota(jnp.int32, (G, NPAD), 0)
    onehot = (batch_ref[:] == ids).astype(jnp.float32)
    sums = jnp.dot(onehot, h_ref[:], preferred_element_type=jnp.float32)
    cnt = jnp.sum(onehot, axis=1, keepdims=True)
    out_ref[:] = sums / jnp.maximum(cnt, 1.0)

  return pl.pallas_call(
      body,
      grid=(1,),
      in_specs=[
          pl.BlockSpec((NPAD, D_OUT), lambda i: (0, 0)),
          pl.BlockSpec((1, NPAD), lambda i: (0, 0)),
      ],
      out_specs=pl.BlockSpec((G, D_OUT), lambda i: (0, 0)),
      out_shape=jax.ShapeDtypeStruct((G, D_OUT), jnp.float32),
  )(h3, batch_row)


@jax.jit
def kernel(x, edge_index, batch, W1, b1, W2, b2, W3, b3):
  src = edge_index[0]
  dst = edge_index[1]
  src2d = jnp.concatenate(
      [src, jnp.zeros((EPAD - E,), jnp.int32)]).reshape(EPAD // 128, 128)
  dst2d = jnp.concatenate(
      [dst, jnp.full((EPAD - E,), N, jnp.int32)]).reshape(EPAD // 128, 128)
  batch_row = jnp.concatenate(
      [batch, jnp.full((NPAD - N,), G, jnp.int32)]).reshape(1, NPAD)
  x_p = jnp.concatenate([x, jnp.zeros((NPAD - N, D_IN), jnp.float32)])

  cnt = _sc_count(dst2d, jnp.zeros((NHALF, 128), jnp.float32),
                  jnp.ones((128, 128), jnp.float32))
  u1, dinv = _tc_first(x_p, W1, cnt)

  W3p = jnp.concatenate([W3, jnp.zeros((D_H, D_H - D_OUT), jnp.float32)],
                        axis=1)
  Ws = jnp.stack([W2, W3p, W3p])
  bs = jnp.stack([b1.reshape(1, D_H), b2.reshape(1, D_H),
                  b2.reshape(1, D_H)])

  def step(carry, wb):
    u, _ = carry
    W_i, b_i = wb
    agg = _sc_aggregate(u, src2d, dst2d)
    u_next = _tc_mid(agg, dinv, b_i, W_i)
    return (u_next, agg), None

  (_, agg3), _ = lax.scan(step, (u1, u1), (Ws, bs))
  h3 = _tc_relu(agg3[:D_OUT // CW], dinv, b3.reshape(1, D_OUT))
  return _tc_finalize(h3, batch_row)


# speedup vs baseline: 2.0887x; 1.0451x over previous
import jax
import jax.numpy as jnp
from jax import lax
from jax.experimental import pallas as pl
from jax.experimental.pallas import tpu as pltpu
from jax.experimental.pallas import tpu_sc as plsc

N = 10000
E = 160000
G = 128
D_IN = 256
D_H = 512
D_OUT = 256

NPAD = 10240
EPAD = 163840
RPT = NPAD // 16
NB = EPAD // 128 // 16
GRP = 2
CW = 128
CN = D_H // CW
NHALF = NPAD // 2
HPT = NHALF // 16
ACCROWS = NHALF + 8
R = 1280

_MESH = plsc.VectorSubcoreMesh(core_axis_name="c", subcore_axis_name="s")


def _sc_count(dst2d, zeros_h, ones_h):
  def body(dst_hbm, z_hbm, ones_hbm, out_hbm, dstv, ones_v, acc):
    c = lax.axis_index("c")
    s = lax.axis_index("s")
    base = c * NHALF
    pltpu.sync_copy(dst_hbm.at[pl.ds(s * NB, NB)], dstv)
    pltpu.sync_copy(ones_hbm, ones_v)
    pltpu.sync_copy(z_hbm.at[pl.ds(s * HPT, HPT)], acc.at[pl.ds(s * HPT, HPT)])

    def remap(j, carry):
      for l in range(8):
        dv = dstv[j, pl.ds(l * 16, 16)] - base
        oob = (dv < 0) | (dv >= NHALF)
        dstv[j, pl.ds(l * 16, 16)] = jnp.where(oob, NHALF, dv)
      return carry

    lax.fori_loop(0, NB, remap, 0)
    plsc.subcore_barrier()

    def step(j, carry):
      pltpu.sync_copy(ones_v, acc.at[dstv.at[j]], add=True)
      return carry

    lax.fori_loop(0, NB, step, 0)
    plsc.subcore_barrier()
    pltpu.sync_copy(acc.at[pl.ds(s * HPT, HPT)],
                    out_hbm.at[pl.ds(base + s * HPT, HPT)])

  kfn = pl.kernel(
      body,
      out_type=jax.ShapeDtypeStruct((NPAD, 128), jnp.float32),
      mesh=_MESH,
      scratch_types=[
          pltpu.VMEM((NB, 128), jnp.int32),
          pltpu.VMEM((128, 128), jnp.float32),
          pltpu.VMEM_SHARED((ACCROWS, 128), jnp.float32),
      ],
  )
  return kfn(dst2d, zeros_h, ones_h)


def _sc_aggregate(u, src2d, dst2d):
  def body(u_hbm, src_hbm, dst_hbm, out_hbm, srcv, dstv, rows, acc, gsem,
           ssem):
    c = lax.axis_index("c")
    s = lax.axis_index("s")
    base = c * NHALF
    pltpu.sync_copy(src_hbm.at[pl.ds(s * NB, NB)], srcv)
    pltpu.sync_copy(dst_hbm.at[pl.ds(s * NB, NB)], dstv)

    def remap(j, carry):
      for l in range(8):
        d = dstv[j, pl.ds(l * 16, 16)] - base
        oob = (d < 0) | (d >= NHALF)
        dstv[j, pl.ds(l * 16, 16)] = jnp.where(oob, NHALF, d)
      return carry

    lax.fori_loop(0, NB, remap, 0)

    NP = NB // (2 * GRP)

    for kk in range(CN):
      pltpu.sync_copy(u_hbm.at[kk, pl.ds(base + s * HPT, HPT)],
                      acc.at[pl.ds(s * HPT, HPT)])
      plsc.subcore_barrier()

      def fire_gather(j0, half):
        for b in range(GRP):
          pltpu.async_copy(u_hbm.at[kk].at[srcv.at[j0 + b]],
                           rows.at[half * GRP + b], gsem)

      def fire_scatter(j0, half):
        for b in range(GRP):
          pltpu.async_copy(rows.at[half * GRP + b],
                           acc.at[dstv.at[j0 + b]], ssem, add=True)

      def drain(sem):
        for b in range(GRP):
          pltpu.make_async_copy(u_hbm.at[kk, pl.ds(0, 128)],
                                rows.at[b], sem).wait()

      fire_gather(0, 0)

      def pair(p, carry):
        j0 = p * 2 * GRP
        drain(gsem)
        fire_gather(j0 + GRP, 1)
        fire_scatter(j0, 0)
        drain(ssem)
        drain(gsem)

        @pl.when(p < NP - 1)
        def _():
          fire_gather(j0 + 2 * GRP, 0)

        fire_scatter(j0 + GRP, 1)
        drain(ssem)
        return carry

      lax.fori_loop(0, NP, pair, 0)
      plsc.subcore_barrier()
      pltpu.sync_copy(acc.at[pl.ds(s * HPT, HPT)],
                      out_hbm.at[kk, pl.ds(base + s * HPT, HPT)])
      plsc.subcore_barrier()

  kfn = pl.kernel(
      body,
      out_type=jax.ShapeDtypeStruct((CN, NPAD, CW), jnp.float32),
      mesh=_MESH,
      scratch_types=[
          pltpu.VMEM((NB, 128), jnp.int32),
          pltpu.VMEM((NB, 128), jnp.int32),
          pltpu.VMEM((2 * GRP, 128, CW), jnp.float32),
          pltpu.VMEM_SHARED((ACCROWS, CW), jnp.float32),
          pltpu.SemaphoreType.DMA,
          pltpu.SemaphoreType.DMA,
      ],
  )
  return kfn(u, src2d, dst2d)


def _tc_first(x_p, W1, cnt2):
  def body(x_ref, w_ref, cnt_ref, u_ref, dinv_ref):
    cnt = cnt_ref[:, :1]
    dinv = lax.rsqrt(cnt + 1.0)
    xw = jnp.dot(x_ref[:], w_ref[:], preferred_element_type=jnp.float32)
    u_ref[0] = xw * dinv
    dinv_ref[:] = dinv

  return pl.pallas_call(
      body,
      grid=(NPAD // R, CN),
      in_specs=[
          pl.BlockSpec((R, D_IN), lambda r, co: (r, 0)),
          pl.BlockSpec((D_IN, CW), lambda r, co: (0, co)),
          pl.BlockSpec((R, 128), lambda r, co: (r, 0)),
      ],
      out_specs=[
          pl.BlockSpec((1, R, CW), lambda r, co: (co, r, 0)),
          pl.BlockSpec((R, 1), lambda r, co: (r, 0)),
      ],
      out_shape=[
          jax.ShapeDtypeStruct((CN, NPAD, CW), jnp.float32),
          jax.ShapeDtypeStruct((NPAD, 1), jnp.float32),
      ],
  )(x_p, W1, cnt2)


def _tc_mid(agg, dinv, b_row, W):
  def body(agg_ref, dinv_ref, b_ref, w_ref, u_ref):
    dinv = dinv_ref[:]
    acc = jnp.zeros((R, CW), dtype=jnp.float32)
    for k2 in range(CN):
      hk = jnp.maximum(agg_ref[k2] * dinv + b_ref[:, k2 * CW:(k2 + 1) * CW],
                       0.0)
      acc += jnp.dot(hk, w_ref[k2 * CW:(k2 + 1) * CW, :],
                     preferred_element_type=jnp.float32)
    u_ref[0] = acc * dinv

  return pl.pallas_call(
      body,
      grid=(NPAD // R, CN),
      in_specs=[
          pl.BlockSpec((CN, R, CW), lambda r, co: (0, r, 0)),
          pl.BlockSpec((R, 1), lambda r, co: (r, 0)),
          pl.BlockSpec((1, D_H), lambda r, co: (0, 0)),
          pl.BlockSpec((D_H, CW), lambda r, co: (0, co)),
      ],
      out_specs=pl.BlockSpec((1, R, CW), lambda r, co: (co, r, 0)),
      out_shape=jax.ShapeDtypeStruct((CN, NPAD, CW), jnp.float32),
  )(agg, dinv, b_row, W)


def _tc_relu(agg, dinv, b_row):
  CO = D_OUT // CW

  def body(agg_ref, dinv_ref, b_ref, h_ref):
    full = jnp.concatenate([agg_ref[k] for k in range(CO)], axis=1)
    h_ref[:] = jnp.maximum(full * dinv_ref[:] + b_ref[:], 0.0)

  return pl.pallas_call(
      body,
      grid=(NPAD // R,),
      in_specs=[
          pl.BlockSpec((CO, R, CW), lambda r: (0, r, 0)),
          pl.BlockSpec((R, 1), lambda r: (r, 0)),
          pl.BlockSpec((1, D_OUT), lambda r: (0, 0)),
      ],
      out_specs=pl.BlockSpec((R, D_OUT), lambda r: (r, 0)),
      out_shape=jax.ShapeDtypeStruct((NPAD, D_OUT), jnp.float32),
  )(agg, dinv, b_row)


def _tc_finalize(h3, batch_row):
  def body(h_ref, batch_ref, out_ref):
    ids = lax.broadcasted_iota(jnp.int32, (G, NPAD), 0)
    onehot = (batch_ref[:] == ids).astype(jnp.float32)
    sums = jnp.dot(onehot, h_ref[:], preferred_element_type=jnp.float32)
    cnt = jnp.sum(onehot, axis=1, keepdims=True)
    out_ref[:] = sums / jnp.maximum(cnt, 1.0)

  return pl.pallas_call(
      body,
      grid=(1,),
      in_specs=[
          pl.BlockSpec((NPAD, D_OUT), lambda i: (0, 0)),
          pl.BlockSpec((1, NPAD), lambda i: (0, 0)),
      ],
      out_specs=pl.BlockSpec((G, D_OUT), lambda i: (0, 0)),
      out_shape=jax.ShapeDtypeStruct((G, D_OUT), jnp.float32),
  )(h3, batch_row)


@jax.jit
def kernel(x, edge_index, batch, W1, b1, W2, b2, W3, b3):
  src = edge_index[0]
  dst = edge_index[1]
  src2d = jnp.concatenate(
      [src, jnp.zeros((EPAD - E,), jnp.int32)]).reshape(EPAD // 128, 128)
  dst2d = jnp.concatenate(
      [dst, jnp.full((EPAD - E,), N, jnp.int32)]).reshape(EPAD // 128, 128)
  batch_row = jnp.concatenate(
      [batch, jnp.full((NPAD - N,), G, jnp.int32)]).reshape(1, NPAD)
  x_p = jnp.concatenate([x, jnp.zeros((NPAD - N, D_IN), jnp.float32)])

  cnt = _sc_count(dst2d, jnp.zeros((NHALF, 128), jnp.float32),
                  jnp.ones((128, 128), jnp.float32))
  u1, dinv = _tc_first(x_p, W1, cnt)

  W3p = jnp.concatenate([W3, jnp.zeros((D_H, D_H - D_OUT), jnp.float32)],
                        axis=1)
  Ws = jnp.stack([W2, W3p, W3p])
  bs = jnp.stack([b1.reshape(1, D_H), b2.reshape(1, D_H),
                  b2.reshape(1, D_H)])

  def step(carry, wb):
    u, _ = carry
    W_i, b_i = wb
    agg = _sc_aggregate(u, src2d, dst2d)
    u_next = _tc_mid(agg, dinv, b_i, W_i)
    return (u_next, agg), None

  (_, agg3), _ = lax.scan(step, (u1, u1), (Ws, bs))
  h3 = _tc_relu(agg3[:D_OUT // CW], dinv, b3.reshape(1, D_OUT))
  return _tc_finalize(h3, batch_row)


# speedup vs baseline: 2.1833x; 1.0453x over previous
import jax
import jax.numpy as jnp
from jax import lax
from jax.experimental import pallas as pl
from jax.experimental.pallas import tpu as pltpu
from jax.experimental.pallas import tpu_sc as plsc

N = 10000
E = 160000
G = 128
D_IN = 256
D_H = 512
D_OUT = 256

NPAD = 10240
EPAD = 163840
RPT = NPAD // 16
NB = EPAD // 128 // 16
GRP = 2
CW = 128
CN = D_H // CW
NHALF = NPAD // 2
HPT = NHALF // 16
ACCROWS = NHALF + 16
R = 1280

_MESH = plsc.VectorSubcoreMesh(core_axis_name="c", subcore_axis_name="s")


def _sc_count(dst2d, zeros_h, ones_h):
  def body(dst_hbm, z_hbm, ones_hbm, out_hbm, dstv, ones_v, acc):
    c = lax.axis_index("c")
    s = lax.axis_index("s")
    base = c * NHALF
    pltpu.sync_copy(dst_hbm.at[pl.ds(s * NB, NB)], dstv)
    pltpu.sync_copy(ones_hbm, ones_v)
    pltpu.sync_copy(z_hbm.at[pl.ds(s * HPT, HPT)], acc.at[pl.ds(s * HPT, HPT)])

    def remap(j, carry):
      for l in range(8):
        dv = dstv[j, pl.ds(l * 16, 16)] - base
        oob = (dv < 0) | (dv >= NHALF)
        dump = NHALF + lax.iota(jnp.int32, 16)
        dstv[j, pl.ds(l * 16, 16)] = jnp.where(oob, dump, dv)
      return carry

    lax.fori_loop(0, NB, remap, 0)
    plsc.subcore_barrier()

    def step(j, carry):
      pltpu.sync_copy(ones_v, acc.at[dstv.at[j]], add=True)
      return carry

    lax.fori_loop(0, NB, step, 0)
    plsc.subcore_barrier()
    pltpu.sync_copy(acc.at[pl.ds(s * HPT, HPT)],
                    out_hbm.at[pl.ds(base + s * HPT, HPT)])

  kfn = pl.kernel(
      body,
      out_type=jax.ShapeDtypeStruct((NPAD, 128), jnp.float32),
      mesh=_MESH,
      scratch_types=[
          pltpu.VMEM((NB, 128), jnp.int32),
          pltpu.VMEM((128, 128), jnp.float32),
          pltpu.VMEM_SHARED((ACCROWS, 128), jnp.float32),
      ],
  )
  return kfn(dst2d, zeros_h, ones_h)


def _sc_aggregate(u, src2d, dst2d):
  def body(u_hbm, src_hbm, dst_hbm, out_hbm, srcv, dstv, rows, acc, gsem,
           ssem):
    c = lax.axis_index("c")
    s = lax.axis_index("s")
    base = c * NHALF
    pltpu.sync_copy(src_hbm.at[pl.ds(s * NB, NB)], srcv)
    pltpu.sync_copy(dst_hbm.at[pl.ds(s * NB, NB)], dstv)

    def remap(j, carry):
      for l in range(8):
        d = dstv[j, pl.ds(l * 16, 16)] - base
        oob = (d < 0) | (d >= NHALF)
        dump = NHALF + lax.iota(jnp.int32, 16)
        dstv[j, pl.ds(l * 16, 16)] = jnp.where(oob, dump, d)
      return carry

    lax.fori_loop(0, NB, remap, 0)

    NP = NB // (2 * GRP)

    for kk in range(CN):
      pltpu.sync_copy(u_hbm.at[kk, pl.ds(base + s * HPT, HPT)],
                      acc.at[pl.ds(s * HPT, HPT)])
      plsc.subcore_barrier()

      def fire_gather(j0, half):
        for b in range(GRP):
          pltpu.async_copy(u_hbm.at[kk].at[srcv.at[j0 + b]],
                           rows.at[half * GRP + b], gsem)

      def fire_scatter(j0, half):
        for b in range(GRP):
          pltpu.async_copy(rows.at[half * GRP + b],
                           acc.at[dstv.at[j0 + b]], ssem, add=True)

      def drain(sem):
        for b in range(GRP):
          pltpu.make_async_copy(u_hbm.at[kk, pl.ds(0, 128)],
                                rows.at[b], sem).wait()

      fire_gather(0, 0)

      def pair(p, carry):
        j0 = p * 2 * GRP
        drain(gsem)
        fire_gather(j0 + GRP, 1)
        fire_scatter(j0, 0)
        drain(ssem)
        drain(gsem)

        @pl.when(p < NP - 1)
        def _():
          fire_gather(j0 + 2 * GRP, 0)

        fire_scatter(j0 + GRP, 1)
        drain(ssem)
        return carry

      lax.fori_loop(0, NP, pair, 0)
      plsc.subcore_barrier()
      pltpu.sync_copy(acc.at[pl.ds(s * HPT, HPT)],
                      out_hbm.at[kk, pl.ds(base + s * HPT, HPT)])
      plsc.subcore_barrier()

  kfn = pl.kernel(
      body,
      out_type=jax.ShapeDtypeStruct((CN, NPAD, CW), jnp.float32),
      mesh=_MESH,
      scratch_types=[
          pltpu.VMEM((NB, 128), jnp.int32),
          pltpu.VMEM((NB, 128), jnp.int32),
          pltpu.VMEM((2 * GRP, 128, CW), jnp.float32),
          pltpu.VMEM_SHARED((ACCROWS, CW), jnp.float32),
          pltpu.SemaphoreType.DMA,
          pltpu.SemaphoreType.DMA,
      ],
  )
  return kfn(u, src2d, dst2d)


def _tc_first(x_p, W1, cnt2):
  def body(x_ref, w_ref, cnt_ref, u_ref, dinv_ref):
    cnt = cnt_ref[:, :1]
    dinv = lax.rsqrt(cnt + 1.0)
    xw = jnp.dot(x_ref[:], w_ref[:], preferred_element_type=jnp.float32)
    u_ref[0] = xw * dinv
    dinv_ref[:] = dinv

  return pl.pallas_call(
      body,
      grid=(NPAD // R, CN),
      in_specs=[
          pl.BlockSpec((R, D_IN), lambda r, co: (r, 0)),
          pl.BlockSpec((D_IN, CW), lambda r, co: (0, co)),
          pl.BlockSpec((R, 128), lambda r, co: (r, 0)),
      ],
      out_specs=[
          pl.BlockSpec((1, R, CW), lambda r, co: (co, r, 0)),
          pl.BlockSpec((R, 1), lambda r, co: (r, 0)),
      ],
      out_shape=[
          jax.ShapeDtypeStruct((CN, NPAD, CW), jnp.float32),
          jax.ShapeDtypeStruct((NPAD, 1), jnp.float32),
      ],
  )(x_p, W1, cnt2)


def _tc_mid(agg, dinv, b_row, W):
  def body(agg_ref, dinv_ref, b_ref, w_ref, u_ref):
    dinv = dinv_ref[:]
    acc = jnp.zeros((R, CW), dtype=jnp.float32)
    for k2 in range(CN):
      hk = jnp.maximum(agg_ref[k2] * dinv + b_ref[:, k2 * CW:(k2 + 1) * CW],
                       0.0)
      acc += jnp.dot(hk, w_ref[k2 * CW:(k2 + 1) * CW, :],
                     preferred_element_type=jnp.float32)
    u_ref[0] = acc * dinv

  return pl.pallas_call(
      body,
      grid=(NPAD // R, CN),
      in_specs=[
          pl.BlockSpec((CN, R, CW), lambda r, co: (0, r, 0)),
          pl.BlockSpec((R, 1), lambda r, co: (r, 0)),
          pl.BlockSpec((1, D_H), lambda r, co: (0, 0)),
          pl.BlockSpec((D_H, CW), lambda r, co: (0, co)),
      ],
      out_specs=pl.BlockSpec((1, R, CW), lambda r, co: (co, r, 0)),
      out_shape=jax.ShapeDtypeStruct((CN, NPAD, CW), jnp.float32),
  )(agg, dinv, b_row, W)


def _tc_relu(agg, dinv, b_row):
  CO = D_OUT // CW

  def body(agg_ref, dinv_ref, b_ref, h_ref):
    full = jnp.concatenate([agg_ref[k] for k in range(CO)], axis=1)
    h_ref[:] = jnp.maximum(full * dinv_ref[:] + b_ref[:], 0.0)

  return pl.pallas_call(
      body,
      grid=(NPAD // R,),
      in_specs=[
          pl.BlockSpec((CO, R, CW), lambda r: (0, r, 0)),
          pl.BlockSpec((R, 1), lambda r: (r, 0)),
          pl.BlockSpec((1, D_OUT), lambda r: (0, 0)),
      ],
      out_specs=pl.BlockSpec((R, D_OUT), lambda r: (r, 0)),
      out_shape=jax.ShapeDtypeStruct((NPAD, D_OUT), jnp.float32),
  )(agg, dinv, b_row)


def _tc_finalize(h3, batch_row):
  def body(h_ref, batch_ref, out_ref):
    ids = lax.broadcasted_iota(jnp.int32, (G, NPAD), 0)
    onehot = (batch_ref[:] == ids).astype(jnp.float32)
    sums = jnp.dot(onehot, h_ref[:], preferred_element_type=jnp.float32)
    cnt = jnp.sum(onehot, axis=1, keepdims=True)
    out_ref[:] = sums / jnp.maximum(cnt, 1.0)

  return pl.pallas_call(
      body,
      grid=(1,),
      in_specs=[
          pl.BlockSpec((NPAD, D_OUT), lambda i: (0, 0)),
          pl.BlockSpec((1, NPAD), lambda i: (0, 0)),
      ],
      out_specs=pl.BlockSpec((G, D_OUT), lambda i: (0, 0)),
      out_shape=jax.ShapeDtypeStruct((G, D_OUT), jnp.float32),
  )(h3, batch_row)


@jax.jit
def kernel(x, edge_index, batch, W1, b1, W2, b2, W3, b3):
  src = edge_index[0]
  dst = edge_index[1]
  src2d = jnp.concatenate(
      [src, jnp.zeros((EPAD - E,), jnp.int32)]).reshape(EPAD // 128, 128)
  dst2d = jnp.concatenate(
      [dst, jnp.full((EPAD - E,), N, jnp.int32)]).reshape(EPAD // 128, 128)
  batch_row = jnp.concatenate(
      [batch, jnp.full((NPAD - N,), G, jnp.int32)]).reshape(1, NPAD)
  x_p = jnp.concatenate([x, jnp.zeros((NPAD - N, D_IN), jnp.float32)])

  cnt = _sc_count(dst2d, jnp.zeros((NHALF, 128), jnp.float32),
                  jnp.ones((128, 128), jnp.float32))
  u1, dinv = _tc_first(x_p, W1, cnt)

  W3p = jnp.concatenate([W3, jnp.zeros((D_H, D_H - D_OUT), jnp.float32)],
                        axis=1)
  Ws = jnp.stack([W2, W3p, W3p])
  bs = jnp.stack([b1.reshape(1, D_H), b2.reshape(1, D_H),
                  b2.reshape(1, D_H)])

  def step(carry, wb):
    u, _ = carry
    W_i, b_i = wb
    agg = _sc_aggregate(u, src2d, dst2d)
    u_next = _tc_mid(agg, dinv, b_i, W_i)
    return (u_next, agg), None

  (_, agg3), _ = lax.scan(step, (u1, u1), (Ws, bs))
  h3 = _tc_relu(agg3[:D_OUT // CW], dinv, b3.reshape(1, D_OUT))
  return _tc_finalize(h3, batch_row)
